# baseline (speedup 1.0000x reference)
import functools

import jax
import jax.numpy as jnp
from jax import lax
from jax.experimental import pallas as pl
from jax.experimental.pallas import tpu as pltpu
from jax.experimental.pallas import tpu_sc as plsc

N_DISC = 10
VOCAB = 1000
DIM = 128
LANES = 16
NB = DIM // LANES

NC = 2
NS = 16
NW = NC * NS

C = 64


def _yc_matmul(x_c, W, b):
    n = x_c.shape[0]
    bm = 2048
    assert n % bm == 0
    b2 = b.reshape(1, DIM)

    def body(x_ref, w_ref, b_ref, o_ref):
        o_ref[...] = (
            jnp.dot(x_ref[...], w_ref[...], preferred_element_type=jnp.float32)
            + b_ref[...]
        )

    return pl.pallas_call(
        body,
        grid=(n // bm,),
        in_specs=[
            pl.BlockSpec((bm, x_c.shape[1]), lambda i: (i, 0)),
            pl.BlockSpec(W.shape, lambda i: (0, 0)),
            pl.BlockSpec((1, DIM), lambda i: (0, 0)),
        ],
        out_specs=pl.BlockSpec((bm, DIM), lambda i: (i, 0)),
        out_shape=jax.ShapeDtypeStruct((n, DIM), jnp.float32),
    )(x_c, W, b2)


def _make_sc_kernel(npad):
    nch_w = npad // (C * NW)
    assert nch_w % 2 == 0
    mesh = plsc.VectorSubcoreMesh(core_axis_name="c", subcore_axis_name="s")

    @functools.partial(
        pl.kernel,
        mesh=mesh,
        compiler_params=pltpu.CompilerParams(use_tc_tiling_on_sc=False, needs_layout_passes=False),
        out_type=jax.ShapeDtypeStruct((npad, DIM), jnp.float32),
        scratch_types=[
            pltpu.VMEM((2, N_DISC, C), jnp.int32),
            pltpu.VMEM((2, N_DISC, C, DIM), jnp.bfloat16),
            pltpu.VMEM((2, C, DIM), jnp.float32),
            pltpu.VMEM((2, C, DIM), jnp.float32),
            pltpu.SemaphoreType.DMA,
            pltpu.SemaphoreType.DMA,
            pltpu.SemaphoreType.DMA,
            pltpu.SemaphoreType.DMA,
            pltpu.SemaphoreType.DMA,
            pltpu.SemaphoreType.DMA,
            pltpu.SemaphoreType.DMA,
            pltpu.SemaphoreType.DMA,
        ],
    )
    def sc(table_hbm, idx_hbm, yc_hbm, out_hbm, idx_v, gat, ybuf, obuf,
           si0, si1, sy0, sy1, sg0, sg1, so0, so1):
        wid = lax.axis_index("s") * NC + lax.axis_index("c")
        first = wid * nch_w
        sem_idx = (si0, si1)
        sem_yc = (sy0, sy1)
        sem_gat = (sg0, sg1)
        sem_out = (so0, so1)

        def idx_cp(chunk, b):
            return pltpu.make_async_copy(
                idx_hbm.at[chunk], idx_v.at[b], sem_idx[b])

        def yc_cp(chunk, b):
            return pltpu.make_async_copy(
                yc_hbm.at[pl.ds(chunk * C, C)], ybuf.at[b], sem_yc[b])

        def gat_cps(b):
            return [
                pltpu.make_async_copy(
                    table_hbm.at[idx_v.at[b, j]], gat.at[b, j], sem_gat[b])
                for j in range(N_DISC)
            ]

        def out_cp(chunk, b):
            return pltpu.make_async_copy(
                obuf.at[b], out_hbm.at[pl.ds(chunk * C, C)], sem_out[b])

        idx_cp(first, 0).start()
        yc_cp(first, 0).start()
        idx_cp(first + 1, 1).start()
        yc_cp(first + 1, 1).start()
        idx_cp(first, 0).wait()
        for cp in gat_cps(0):
            cp.start()

        def pair_body(g, carry):
            for b in range(2):
                ci = 2 * g + b
                chunk = first + ci
                o = 1 - b
                @pl.when(ci + 1 < nch_w)
                def _():
                    idx_cp(chunk + 1, o).wait()
                    for cp in gat_cps(o):
                        cp.start()

                yc_cp(chunk, b).wait()
                for cp in gat_cps(b):
                    cp.wait()

                @pl.when(ci + 2 < nch_w)
                def _():
                    idx_cp(chunk + 2, b).start()

                @pl.when(ci >= 2)
                def _():
                    out_cp(chunk - 2, b).wait()

                def node_body(n, c2):
                    acc = [ybuf[b, n, pl.ds(d * LANES, LANES)]
                           for d in range(NB)]
                    for j in range(N_DISC):
                        for g in range(4):
                            w = plsc.bitcast(
                                gat[b, j, n, pl.ds(g * 32, 32)], jnp.int32)
                            acc[g] = acc[g] + plsc.bitcast(
                                lax.shift_left(w, 16), jnp.float32)
                            acc[4 + g] = acc[4 + g] + plsc.bitcast(
                                w & jnp.int32(-65536), jnp.float32)
                    for d in range(NB):
                        obuf[b, n, pl.ds(d * LANES, LANES)] = jnp.maximum(
                            acc[d], 0.0)
                    return c2

                lax.fori_loop(0, C, node_body, 0)
                out_cp(chunk, b).start()

                @pl.when(ci + 2 < nch_w)
                def _():
                    yc_cp(chunk + 2, b).start()
            return carry

        lax.fori_loop(0, nch_w // 2, pair_body, 0)
        out_cp(first + nch_w - 2, 0).wait()
        out_cp(first + nch_w - 1, 1).wait()

    return sc


def kernel(x_d, x_c, emb_tables, W, b):
    n = x_d.shape[0]
    step = 2 * C * NW
    npad = -(-n // step) * step
    pad = npad - n

    half = DIM // 2
    perm = jnp.stack(
        [jnp.arange(half, dtype=jnp.int32),
         jnp.arange(half, dtype=jnp.int32) + half], axis=1).reshape(-1)
    table = emb_tables.reshape(N_DISC * VOCAB, DIM)[:, perm]
    table = table.astype(jnp.bfloat16)
    offs = (jnp.arange(N_DISC, dtype=jnp.int32) * VOCAB)[None, :]
    flat = x_d.astype(jnp.int32) + offs
    flat = jnp.pad(flat, ((0, pad), (0, 0)))
    idx3 = flat.reshape(npad // C, C, N_DISC).transpose(0, 2, 1)

    x_c_pad = jnp.pad(x_c, ((0, pad), (0, 0)))
    yc = _yc_matmul(x_c_pad, W, b)

    out = _make_sc_kernel(npad)(table, idx3, yc)
    return out[:n]


# speedup vs baseline: 1.2722x; 1.2722x over previous
import functools

import jax
import jax.numpy as jnp
from jax import lax
from jax.experimental import pallas as pl
from jax.experimental.pallas import tpu as pltpu
from jax.experimental.pallas import tpu_sc as plsc

N_DISC = 10
VOCAB = 1000
DIM = 128
LANES = 16
NB = DIM // LANES

NC = 2
NS = 16
NW = NC * NS

C = 48


def _yc_matmul(x_c, W, b):
    n = x_c.shape[0]
    bm = 512
    assert n % bm == 0
    b2 = b.reshape(1, DIM)

    def body(x_ref, w_ref, b_ref, o_ref):
        o_ref[...] = (
            jnp.dot(x_ref[...], w_ref[...], preferred_element_type=jnp.float32)
            + b_ref[...]
        )

    return pl.pallas_call(
        body,
        grid=(n // bm,),
        in_specs=[
            pl.BlockSpec((bm, x_c.shape[1]), lambda i: (i, 0)),
            pl.BlockSpec(W.shape, lambda i: (0, 0)),
            pl.BlockSpec((1, DIM), lambda i: (0, 0)),
        ],
        out_specs=pl.BlockSpec((bm, DIM), lambda i: (i, 0)),
        out_shape=jax.ShapeDtypeStruct((n, DIM), jnp.float32),
    )(x_c, W, b2)


def _make_sc_kernel(npad):
    nch_w = npad // (C * NW)
    assert nch_w % 2 == 0
    mesh = plsc.VectorSubcoreMesh(core_axis_name="c", subcore_axis_name="s")

    @functools.partial(
        pl.kernel,
        mesh=mesh,
        compiler_params=pltpu.CompilerParams(use_tc_tiling_on_sc=False, needs_layout_passes=False),
        out_type=jax.ShapeDtypeStruct((npad, DIM), jnp.float32),
        scratch_types=[
            pltpu.VMEM((2, N_DISC, C), jnp.int32),
            pltpu.VMEM((2, N_DISC, C, DIM), jnp.bfloat16),
            pltpu.VMEM((2, C, DIM), jnp.float32),
            pltpu.VMEM((2, C, DIM), jnp.float32),
            pltpu.SemaphoreType.DMA,
            pltpu.SemaphoreType.DMA,
            pltpu.SemaphoreType.DMA,
            pltpu.SemaphoreType.DMA,
            pltpu.SemaphoreType.DMA,
            pltpu.SemaphoreType.DMA,
            pltpu.SemaphoreType.DMA,
            pltpu.SemaphoreType.DMA,
        ],
    )
    def sc(table_hbm, idx_hbm, yc_hbm, out_hbm, idx_v, gat, ybuf, obuf,
           si0, si1, sy0, sy1, sg0, sg1, so0, so1):
        wid = lax.axis_index("s") * NC + lax.axis_index("c")
        first = wid * nch_w
        sem_idx = (si0, si1)
        sem_yc = (sy0, sy1)
        sem_gat = (sg0, sg1)
        sem_out = (so0, so1)

        def idx_cp(chunk, b):
            return pltpu.make_async_copy(
                idx_hbm.at[chunk], idx_v.at[b], sem_idx[b])

        def yc_cp(chunk, b):
            return pltpu.make_async_copy(
                yc_hbm.at[pl.ds(chunk * C, C)], ybuf.at[b], sem_yc[b])

        def gat_cps(b):
            return [
                pltpu.make_async_copy(
                    table_hbm.at[idx_v.at[b, j]], gat.at[b, j], sem_gat[b])
                for j in range(N_DISC)
            ]

        def out_cp(chunk, b):
            return pltpu.make_async_copy(
                obuf.at[b], out_hbm.at[pl.ds(chunk * C, C)], sem_out[b])

        idx_cp(first, 0).start()
        yc_cp(first, 0).start()
        idx_cp(first + 1, 1).start()
        yc_cp(first + 1, 1).start()
        idx_cp(first, 0).wait()
        for cp in gat_cps(0):
            cp.start()

        def pair_body(g, carry):
            for b in range(2):
                ci = 2 * g + b
                chunk = first + ci
                o = 1 - b
                @pl.when(ci + 1 < nch_w)
                def _():
                    idx_cp(chunk + 1, o).wait()
                    for cp in gat_cps(o):
                        cp.start()

                yc_cp(chunk, b).wait()
                for cp in gat_cps(b):
                    cp.wait()

                @pl.when(ci + 2 < nch_w)
                def _():
                    idx_cp(chunk + 2, b).start()

                @pl.when(ci >= 2)
                def _():
                    out_cp(chunk - 2, b).wait()

                def node_body(n, c2):
                    acc = [ybuf[b, n, pl.ds(d * LANES, LANES)]
                           for d in range(NB)]
                    for j in range(N_DISC):
                        for g in range(4):
                            w = plsc.bitcast(
                                gat[b, j, n, pl.ds(g * 32, 32)], jnp.int32)
                            acc[g] = acc[g] + plsc.bitcast(
                                lax.shift_left(w, 16), jnp.float32)
                            acc[4 + g] = acc[4 + g] + plsc.bitcast(
                                w & jnp.int32(-65536), jnp.float32)
                    for d in range(NB):
                        obuf[b, n, pl.ds(d * LANES, LANES)] = jnp.maximum(
                            acc[d], 0.0)
                    return c2

                lax.fori_loop(0, C, node_body, 0)
                out_cp(chunk, b).start()

                @pl.when(ci + 2 < nch_w)
                def _():
                    yc_cp(chunk + 2, b).start()
            return carry

        lax.fori_loop(0, nch_w // 2, pair_body, 0)
        out_cp(first + nch_w - 2, 0).wait()
        out_cp(first + nch_w - 1, 1).wait()

    return sc


def kernel(x_d, x_c, emb_tables, W, b):
    n = x_d.shape[0]
    step = 2 * C * NW
    npad = -(-n // step) * step
    pad = npad - n

    half = DIM // 2
    perm = jnp.stack(
        [jnp.arange(half, dtype=jnp.int32),
         jnp.arange(half, dtype=jnp.int32) + half], axis=1).reshape(-1)
    table = emb_tables.reshape(N_DISC * VOCAB, DIM)[:, perm]
    table = table.astype(jnp.bfloat16)
    offs = (jnp.arange(N_DISC, dtype=jnp.int32) * VOCAB)[None, :]
    flat = x_d.astype(jnp.int32) + offs
    flat = jnp.pad(flat, ((0, pad), (0, 0)))
    idx3 = flat.reshape(npad // C, C, N_DISC).transpose(0, 2, 1)

    x_c_pad = jnp.pad(x_c, ((0, pad), (0, 0)))
    yc = _yc_matmul(x_c_pad, W, b)

    out = _make_sc_kernel(npad)(table, idx3, yc)
    return out[:n]


# speedup vs baseline: 2.4950x; 1.9612x over previous
import functools

import jax
import jax.numpy as jnp
from jax import lax
from jax.experimental import pallas as pl
from jax.experimental.pallas import tpu as pltpu
from jax.experimental.pallas import tpu_sc as plsc

N_DISC = 10
VOCAB = 1000
DIM = 128
LANES = 16
NB = DIM // LANES

NC = 2
NS = 16
NW = NC * NS

C = 32


def _yc_matmul(x_c, W, b):
    n = x_c.shape[0]
    bm = 512
    assert n % bm == 0
    b2 = b.reshape(1, DIM)

    def body(x_ref, w_ref, b_ref, o_ref):
        o_ref[...] = (
            jnp.dot(x_ref[...], w_ref[...], preferred_element_type=jnp.float32)
            + b_ref[...]
        )

    return pl.pallas_call(
        body,
        grid=(n // bm,),
        in_specs=[
            pl.BlockSpec((bm, x_c.shape[1]), lambda i: (i, 0)),
            pl.BlockSpec(W.shape, lambda i: (0, 0)),
            pl.BlockSpec((1, DIM), lambda i: (0, 0)),
        ],
        out_specs=pl.BlockSpec((bm, DIM), lambda i: (i, 0)),
        out_shape=jax.ShapeDtypeStruct((n, DIM), jnp.float32),
    )(x_c, W, b2)


def _make_sc_kernel(npad):
    nch_w = npad // (C * NW)
    assert nch_w % 2 == 0
    mesh = plsc.VectorSubcoreMesh(core_axis_name="c", subcore_axis_name="s")

    @functools.partial(
        pl.kernel,
        mesh=mesh,
        compiler_params=pltpu.CompilerParams(use_tc_tiling_on_sc=False, needs_layout_passes=False),
        out_type=jax.ShapeDtypeStruct((npad, DIM), jnp.float32),
        scratch_types=[
            pltpu.VMEM_SHARED((N_DISC * VOCAB, DIM), jnp.bfloat16),
            pltpu.VMEM((2, N_DISC, C), jnp.int32),
            pltpu.VMEM((2, N_DISC, C, DIM), jnp.bfloat16),
            pltpu.VMEM((2, C, DIM), jnp.float32),
            pltpu.VMEM((2, C, DIM), jnp.float32),
            pltpu.SemaphoreType.DMA,
            pltpu.SemaphoreType.DMA,
            pltpu.SemaphoreType.DMA,
            pltpu.SemaphoreType.DMA,
            pltpu.SemaphoreType.DMA,
            pltpu.SemaphoreType.DMA,
            pltpu.SemaphoreType.DMA,
            pltpu.SemaphoreType.DMA,
        ],
    )
    def sc(table_hbm, idx_hbm, yc_hbm, out_hbm, tbl_sp, idx_v, gat, ybuf,
           obuf, si0, si1, sy0, sy1, sg0, sg1, so0, so1):
        wid = lax.axis_index("s") * NC + lax.axis_index("c")
        first = wid * nch_w

        @pl.when(lax.axis_index("s") == 0)
        def _():
            pltpu.sync_copy(table_hbm, tbl_sp)

        plsc.subcore_barrier()
        sem_idx = (si0, si1)
        sem_yc = (sy0, sy1)
        sem_gat = (sg0, sg1)
        sem_out = (so0, so1)

        def idx_cp(chunk, b):
            return pltpu.make_async_copy(
                idx_hbm.at[chunk], idx_v.at[b], sem_idx[b])

        def yc_cp(chunk, b):
            return pltpu.make_async_copy(
                yc_hbm.at[pl.ds(chunk * C, C)], ybuf.at[b], sem_yc[b])

        def gat_cps(b):
            return [
                pltpu.make_async_copy(
                    tbl_sp.at[idx_v.at[b, j]], gat.at[b, j], sem_gat[b])
                for j in range(N_DISC)
            ]

        def out_cp(chunk, b):
            return pltpu.make_async_copy(
                obuf.at[b], out_hbm.at[pl.ds(chunk * C, C)], sem_out[b])

        idx_cp(first, 0).start()
        yc_cp(first, 0).start()
        idx_cp(first + 1, 1).start()
        yc_cp(first + 1, 1).start()
        idx_cp(first, 0).wait()
        for cp in gat_cps(0):
            cp.start()

        def pair_body(g, carry):
            for b in range(2):
                ci = 2 * g + b
                chunk = first + ci
                o = 1 - b
                @pl.when(ci + 1 < nch_w)
                def _():
                    idx_cp(chunk + 1, o).wait()
                    for cp in gat_cps(o):
                        cp.start()

                yc_cp(chunk, b).wait()
                for cp in gat_cps(b):
                    cp.wait()

                @pl.when(ci + 2 < nch_w)
                def _():
                    idx_cp(chunk + 2, b).start()

                @pl.when(ci >= 2)
                def _():
                    out_cp(chunk - 2, b).wait()

                def node_body(n, c2):
                    acc = [ybuf[b, n, pl.ds(d * LANES, LANES)]
                           for d in range(NB)]
                    for j in range(N_DISC):
                        for g in range(4):
                            w = plsc.bitcast(
                                gat[b, j, n, pl.ds(g * 32, 32)], jnp.int32)
                            acc[g] = acc[g] + plsc.bitcast(
                                lax.shift_left(w, 16), jnp.float32)
                            acc[4 + g] = acc[4 + g] + plsc.bitcast(
                                w & jnp.int32(-65536), jnp.float32)
                    for d in range(NB):
                        obuf[b, n, pl.ds(d * LANES, LANES)] = jnp.maximum(
                            acc[d], 0.0)
                    return c2

                lax.fori_loop(0, C, node_body, 0)
                out_cp(chunk, b).start()

                @pl.when(ci + 2 < nch_w)
                def _():
                    yc_cp(chunk + 2, b).start()
            return carry

        lax.fori_loop(0, nch_w // 2, pair_body, 0)
        out_cp(first + nch_w - 2, 0).wait()
        out_cp(first + nch_w - 1, 1).wait()

    return sc


def kernel(x_d, x_c, emb_tables, W, b):
    n = x_d.shape[0]
    npad = -(-n // (C * NW)) * (C * NW)
    pad = npad - n

    half = DIM // 2
    perm = jnp.stack(
        [jnp.arange(half, dtype=jnp.int32),
         jnp.arange(half, dtype=jnp.int32) + half], axis=1).reshape(-1)
    table = emb_tables.reshape(N_DISC * VOCAB, DIM)[:, perm]
    table = table.astype(jnp.bfloat16)
    offs = (jnp.arange(N_DISC, dtype=jnp.int32) * VOCAB)[None, :]
    flat = x_d.astype(jnp.int32) + offs
    flat = jnp.pad(flat, ((0, pad), (0, 0)))
    idx3 = flat.reshape(npad // C, C, N_DISC).transpose(0, 2, 1)

    x_c_pad = jnp.pad(x_c, ((0, pad), (0, 0)))
    yc = _yc_matmul(x_c_pad, W, b)

    out = _make_sc_kernel(npad)(table, idx3, yc)
    return out[:n]


# speedup vs baseline: 3.9993x; 1.6029x over previous
import functools

import jax
import jax.numpy as jnp
from jax import lax
from jax.experimental import pallas as pl
from jax.experimental.pallas import tpu as pltpu
from jax.experimental.pallas import tpu_sc as plsc

N_DISC = 10
VOCAB = 1000
DIM = 128
LANES = 16
NB = DIM // LANES

NC = 2
NS = 16
NW = NC * NS

C = 32


def _yc_matmul(x_ct, W, b):
    n = x_ct.shape[1]
    bm = 4096
    grid = -(-n // bm)
    b2 = b.reshape(1, DIM)

    def body(x_ref, w_ref, b_ref, o_ref):
        o_ref[...] = lax.dot_general(
            x_ref[...], w_ref[...], (((0,), (0,)), ((), ())),
            preferred_element_type=jnp.float32,
        ) + b_ref[...]

    return pl.pallas_call(
        body,
        grid=(grid,),
        in_specs=[
            pl.BlockSpec((x_ct.shape[0], bm), lambda i: (0, i)),
            pl.BlockSpec(W.shape, lambda i: (0, 0)),
            pl.BlockSpec((1, DIM), lambda i: (0, 0)),
        ],
        out_specs=pl.BlockSpec((bm, DIM), lambda i: (i, 0)),
        out_shape=jax.ShapeDtypeStruct((n, DIM), jnp.float32),
    )(x_ct, W, b2)


def _make_sc_kernel(npad, n):
    nch_w = npad // (C * NW)
    nreal = n // C
    assert nch_w % 2 == 0 and n % C == 0
    mesh = plsc.VectorSubcoreMesh(core_axis_name="c", subcore_axis_name="s")

    @functools.partial(
        pl.kernel,
        mesh=mesh,
        compiler_params=pltpu.CompilerParams(use_tc_tiling_on_sc=False, needs_layout_passes=False),
        out_type=jax.ShapeDtypeStruct((n, DIM), jnp.float32),
        scratch_types=[
            pltpu.VMEM_SHARED((N_DISC * VOCAB, DIM), jnp.bfloat16),
            pltpu.VMEM((2, N_DISC, C), jnp.int32),
            pltpu.VMEM((2, N_DISC, C, DIM), jnp.bfloat16),
            pltpu.VMEM((2, C, DIM), jnp.float32),
            pltpu.VMEM((2, C, DIM), jnp.float32),
            pltpu.SemaphoreType.DMA,
            pltpu.SemaphoreType.DMA,
            pltpu.SemaphoreType.DMA,
            pltpu.SemaphoreType.DMA,
            pltpu.SemaphoreType.DMA,
            pltpu.SemaphoreType.DMA,
            pltpu.SemaphoreType.DMA,
            pltpu.SemaphoreType.DMA,
        ],
    )
    def sc(table_hbm, idx_hbm, yc_hbm, out_hbm, tbl_sp, idx_v, gat, ybuf,
           obuf, si0, si1, sy0, sy1, sg0, sg1, so0, so1):
        wid = lax.axis_index("s") * NC + lax.axis_index("c")
        first = wid * nch_w

        @pl.when(lax.axis_index("s") == 0)
        def _():
            pltpu.sync_copy(table_hbm, tbl_sp)

        plsc.subcore_barrier()
        sem_idx = (si0, si1)
        sem_yc = (sy0, sy1)
        sem_gat = (sg0, sg1)
        sem_out = (so0, so1)

        def idx_cp(chunk, b):
            return pltpu.make_async_copy(
                idx_hbm.at[chunk], idx_v.at[b], sem_idx[b])

        def yc_cp(chunk, b):
            return pltpu.make_async_copy(
                yc_hbm.at[pl.ds(chunk * C, C)], ybuf.at[b], sem_yc[b])

        def gat_cps(b):
            return [
                pltpu.make_async_copy(
                    tbl_sp.at[idx_v.at[b, j]], gat.at[b, j], sem_gat[b])
                for j in range(N_DISC)
            ]

        def out_cp(chunk, b):
            return pltpu.make_async_copy(
                obuf.at[b], out_hbm.at[pl.ds(chunk * C, C)], sem_out[b])

        idx_cp(first, 0).start()
        yc_cp(first, 0).start()
        idx_cp(first + 1, 1).start()
        yc_cp(first + 1, 1).start()
        idx_cp(first, 0).wait()
        for cp in gat_cps(0):
            cp.start()

        def pair_body(g, carry):
            for b in range(2):
                ci = 2 * g + b
                chunk = first + ci
                o = 1 - b
                @pl.when(ci + 1 < nch_w)
                def _():
                    idx_cp(chunk + 1, o).wait()
                    for cp in gat_cps(o):
                        cp.start()

                @pl.when(chunk < nreal)
                def _():
                    yc_cp(chunk, b).wait()
                for cp in gat_cps(b):
                    cp.wait()

                @pl.when(ci + 2 < nch_w)
                def _():
                    idx_cp(chunk + 2, b).start()

                @pl.when(jnp.logical_and(ci >= 2, chunk - 2 < nreal))
                def _():
                    out_cp(chunk - 2, b).wait()

                def node_body(n, c2):
                    acc = [ybuf[b, n, pl.ds(d * LANES, LANES)]
                           for d in range(NB)]
                    for j in range(N_DISC):
                        for g in range(4):
                            w = plsc.bitcast(
                                gat[b, j, n, pl.ds(g * 32, 32)], jnp.int32)
                            acc[g] = acc[g] + plsc.bitcast(
                                lax.shift_left(w, 16), jnp.float32)
                            acc[4 + g] = acc[4 + g] + plsc.bitcast(
                                w & jnp.int32(-65536), jnp.float32)
                    for d in range(NB):
                        obuf[b, n, pl.ds(d * LANES, LANES)] = jnp.maximum(
                            acc[d], 0.0)
                    return c2

                lax.fori_loop(0, C, node_body, 0)

                @pl.when(chunk < nreal)
                def _():
                    out_cp(chunk, b).start()

                @pl.when(jnp.logical_and(ci + 2 < nch_w, chunk + 2 < nreal))
                def _():
                    yc_cp(chunk + 2, b).start()
            return carry

        lax.fori_loop(0, nch_w // 2, pair_body, 0)
        @pl.when(first + nch_w - 2 < nreal)
        def _():
            out_cp(first + nch_w - 2, 0).wait()

        @pl.when(first + nch_w - 1 < nreal)
        def _():
            out_cp(first + nch_w - 1, 1).wait()

    return sc


def kernel(x_d, x_c, emb_tables, W, b):
    n = x_d.shape[0]
    npad = -(-n // (C * NW)) * (C * NW)
    pad = npad - n

    half = DIM // 2
    perm = jnp.stack(
        [jnp.arange(half, dtype=jnp.int32),
         jnp.arange(half, dtype=jnp.int32) + half], axis=1).reshape(-1)
    table = emb_tables.reshape(N_DISC * VOCAB, DIM)[:, perm]
    table = table.astype(jnp.bfloat16)
    offs = (jnp.arange(N_DISC, dtype=jnp.int32) * VOCAB)[None, :]
    flat = x_d.astype(jnp.int32) + offs
    flat = jnp.pad(flat, ((0, pad), (0, 0)))
    idx3 = flat.reshape(npad // C, C, N_DISC).transpose(0, 2, 1)

    yc = _yc_matmul(x_c.T, W, b)

    return _make_sc_kernel(npad, n)(table, idx3, yc)


# speedup vs baseline: 4.0041x; 1.0012x over previous
import functools

import jax
import jax.numpy as jnp
from jax import lax
from jax.experimental import pallas as pl
from jax.experimental.pallas import tpu as pltpu
from jax.experimental.pallas import tpu_sc as plsc

N_DISC = 10
VOCAB = 1000
DIM = 128
LANES = 16
NB = DIM // LANES

NC = 2
NS = 16
NW = NC * NS

C = 32


def _yc_matmul(x_ct, W, b):
    n = x_ct.shape[1]
    bm = 4096
    grid = -(-n // bm)
    b2 = b.reshape(1, DIM)

    def body(x_ref, w_ref, b_ref, o_ref):
        o_ref[...] = lax.dot_general(
            x_ref[...], w_ref[...], (((0,), (0,)), ((), ())),
            preferred_element_type=jnp.float32,
        ) + b_ref[...]

    return pl.pallas_call(
        body,
        grid=(grid,),
        in_specs=[
            pl.BlockSpec((x_ct.shape[0], bm), lambda i: (0, i)),
            pl.BlockSpec(W.shape, lambda i: (0, 0)),
            pl.BlockSpec((1, DIM), lambda i: (0, 0)),
        ],
        out_specs=pl.BlockSpec((bm, DIM), lambda i: (i, 0)),
        out_shape=jax.ShapeDtypeStruct((n, DIM), jnp.float32),
    )(x_ct, W, b2)


def _make_sc_kernel(n):
    nreal = n // C
    nch_w = -(-nreal // NW)
    nch_w += nch_w % 2
    assert n % C == 0
    mesh = plsc.VectorSubcoreMesh(core_axis_name="c", subcore_axis_name="s")

    @functools.partial(
        pl.kernel,
        mesh=mesh,
        compiler_params=pltpu.CompilerParams(use_tc_tiling_on_sc=False, needs_layout_passes=False),
        out_type=jax.ShapeDtypeStruct((n, DIM), jnp.float32),
        scratch_types=[
            pltpu.VMEM_SHARED((N_DISC * VOCAB, DIM), jnp.bfloat16),
            pltpu.VMEM((2, N_DISC, C), jnp.int32),
            pltpu.VMEM((2, N_DISC, C, DIM), jnp.bfloat16),
            pltpu.VMEM((2, C, DIM), jnp.float32),
            pltpu.VMEM((2, C, DIM), jnp.float32),
            pltpu.SemaphoreType.DMA,
            pltpu.SemaphoreType.DMA,
            pltpu.SemaphoreType.DMA,
            pltpu.SemaphoreType.DMA,
            pltpu.SemaphoreType.DMA,
            pltpu.SemaphoreType.DMA,
            pltpu.SemaphoreType.DMA,
            pltpu.SemaphoreType.DMA,
        ],
    )
    def sc(table_hbm, idx_hbm, yc_hbm, out_hbm, tbl_sp, idx_v, gat, ybuf,
           obuf, si0, si1, sy0, sy1, sg0, sg1, so0, so1):
        wid = lax.axis_index("s") * NC + lax.axis_index("c")
        first = wid * nch_w

        @pl.when(lax.axis_index("s") == 0)
        def _():
            pltpu.sync_copy(table_hbm, tbl_sp)

        plsc.subcore_barrier()
        sem_idx = (si0, si1)
        sem_yc = (sy0, sy1)
        sem_gat = (sg0, sg1)
        sem_out = (so0, so1)

        def idx_cp(chunk, b):
            return pltpu.make_async_copy(
                idx_hbm.at[chunk], idx_v.at[b], sem_idx[b])

        def yc_cp(chunk, b):
            return pltpu.make_async_copy(
                yc_hbm.at[pl.ds(chunk * C, C)], ybuf.at[b], sem_yc[b])

        def gat_cps(b):
            return [
                pltpu.make_async_copy(
                    tbl_sp.at[idx_v.at[b, j]], gat.at[b, j], sem_gat[b])
                for j in range(N_DISC)
            ]

        def out_cp(chunk, b):
            return pltpu.make_async_copy(
                obuf.at[b], out_hbm.at[pl.ds(chunk * C, C)], sem_out[b])

        idx_cp(first, 0).start()
        yc_cp(first, 0).start()
        idx_cp(first + 1, 1).start()
        yc_cp(first + 1, 1).start()
        idx_cp(first, 0).wait()
        for cp in gat_cps(0):
            cp.start()

        def pair_body(g, carry):
            for b in range(2):
                ci = 2 * g + b
                chunk = first + ci
                o = 1 - b
                @pl.when(jnp.logical_and(ci + 1 < nch_w, chunk + 1 < nreal))
                def _():
                    idx_cp(chunk + 1, o).wait()
                    for cp in gat_cps(o):
                        cp.start()

                @pl.when(chunk < nreal)
                def _():
                    yc_cp(chunk, b).wait()
                    for cp in gat_cps(b):
                        cp.wait()

                @pl.when(jnp.logical_and(ci + 2 < nch_w,
                                         chunk + 2 < nreal))
                def _():
                    idx_cp(chunk + 2, b).start()

                @pl.when(jnp.logical_and(ci >= 2, chunk - 2 < nreal))
                def _():
                    out_cp(chunk - 2, b).wait()

                def node_body(n, c2):
                    acc = [ybuf[b, n, pl.ds(d * LANES, LANES)]
                           for d in range(NB)]
                    for j in range(N_DISC):
                        for g in range(4):
                            w = plsc.bitcast(
                                gat[b, j, n, pl.ds(g * 32, 32)], jnp.int32)
                            acc[g] = acc[g] + plsc.bitcast(
                                lax.shift_left(w, 16), jnp.float32)
                            acc[4 + g] = acc[4 + g] + plsc.bitcast(
                                w & jnp.int32(-65536), jnp.float32)
                    for d in range(NB):
                        obuf[b, n, pl.ds(d * LANES, LANES)] = jnp.maximum(
                            acc[d], 0.0)
                    return c2

                lax.fori_loop(0, C, node_body, 0)

                @pl.when(chunk < nreal)
                def _():
                    out_cp(chunk, b).start()

                @pl.when(jnp.logical_and(ci + 2 < nch_w, chunk + 2 < nreal))
                def _():
                    yc_cp(chunk + 2, b).start()
            return carry

        lax.fori_loop(0, nch_w // 2, pair_body, 0)
        @pl.when(first + nch_w - 2 < nreal)
        def _():
            out_cp(first + nch_w - 2, 0).wait()

        @pl.when(first + nch_w - 1 < nreal)
        def _():
            out_cp(first + nch_w - 1, 1).wait()

    return sc


def kernel(x_d, x_c, emb_tables, W, b):
    n = x_d.shape[0]

    half = DIM // 2
    perm = jnp.stack(
        [jnp.arange(half, dtype=jnp.int32),
         jnp.arange(half, dtype=jnp.int32) + half], axis=1).reshape(-1)
    table = emb_tables.reshape(N_DISC * VOCAB, DIM)[:, perm]
    table = table.astype(jnp.bfloat16)
    offs = (jnp.arange(N_DISC, dtype=jnp.int32) * VOCAB)[:, None]
    flat_t = x_d.T.astype(jnp.int32) + offs
    idx3 = flat_t.reshape(N_DISC, n // C, C).transpose(1, 0, 2)

    yc = _yc_matmul(x_c.T, W, b)

    return _make_sc_kernel(n)(table, idx3, yc)
